# baseline (speedup 1.0000x reference)
import functools

import jax
import jax.numpy as jnp
from jax import lax
from jax.experimental import pallas as pl
from jax.experimental.pallas import tpu as pltpu
from jax.experimental.pallas import tpu_sc as plsc

D = 128
LANES = 16
NC = 2
NS = 16
NW = NC * NS
B = 128
CCOLS = 16
CROWS = 10240
SPLIT0 = 0.46


def _acc_body(nb0, base1, start1, n8, rag, rows_per_tile, x_hbm, eidx_hbm,
              accp_hbm, src_v, dst_v, rowbuf, acc_sh, gsem):
    core = lax.axis_index("c")
    sub = lax.axis_index("s")

    s0 = sub * nb0
    s1 = start1 + sub * base1 + 8 * jnp.minimum(sub, n8)

    @pl.when(core == 0)
    def _():
        pltpu.sync_copy(eidx_hbm.at[0, pl.ds(s0, nb0)], src_v.at[pl.ds(0, nb0)])
        pltpu.sync_copy(eidx_hbm.at[1, pl.ds(s0, nb0)], dst_v.at[pl.ds(0, nb0)])

    @pl.when(core == 1)
    def _():
        pltpu.sync_copy(eidx_hbm.at[0, pl.ds(s1, base1)],
                        src_v.at[pl.ds(0, base1)])
        pltpu.sync_copy(eidx_hbm.at[1, pl.ds(s1, base1)],
                        dst_v.at[pl.ds(0, base1)])

    @pl.when((core == 1) & (sub < n8))
    def _():
        pltpu.sync_copy(eidx_hbm.at[0, pl.ds(s1 + base1, 8)],
                        src_v.at[pl.ds(base1, 8)])
        pltpu.sync_copy(eidx_hbm.at[1, pl.ds(s1 + base1, 8)],
                        dst_v.at[pl.ds(base1, 8)])

    if rag:
        @pl.when((core == 1) & (sub == NS - 1))
        def _():
            pltpu.sync_copy(eidx_hbm.at[0, pl.ds(s1 + base1, rag)],
                            src_v.at[pl.ds(base1, rag)])
            pltpu.sync_copy(eidx_hbm.at[1, pl.ds(s1 + base1, rag)],
                            dst_v.at[pl.ds(base1, rag)])

    nb_w = jnp.where(
        core == 0, nb0,
        base1 + 8 * (sub < n8).astype(jnp.int32)
        + rag * (sub == NS - 1).astype(jnp.int32))

    zeros = jnp.zeros((LANES,), jnp.float32)

    def _zrow(i, carry):
        for j in range(D // LANES):
            rowbuf[i, pl.ds(j * LANES, LANES)] = zeros
        return carry
    lax.fori_loop(0, B, _zrow, 0)

    base = sub * rows_per_tile
    nfull = rows_per_tile // B
    rem = rows_per_tile - nfull * B
    for r in range(nfull):
        pltpu.sync_copy(rowbuf, acc_sh.at[pl.ds(base + r * B, B)])
    if rem:
        pltpu.sync_copy(rowbuf.at[pl.ds(0, rem)],
                        acc_sh.at[pl.ds(base + nfull * B, rem)])
    plsc.subcore_barrier()

    def _batch(b, carry):
        pltpu.async_copy(x_hbm.at[src_v.at[b]], rowbuf, gsem).wait()
        pltpu.sync_copy(rowbuf, acc_sh.at[dst_v.at[b]], add=True)
        return carry
    lax.fori_loop(0, nb_w, _batch, 0)
    plsc.subcore_barrier()

    pltpu.sync_copy(acc_sh.at[pl.ds(base, rows_per_tile)],
                    accp_hbm.at[core, pl.ds(base, rows_per_tile)])


def _cnt_body(nbc, n8c, ragc, tok_hbm, c1_hbm, c2_hbm, cp_hbm, c1_v, c2_v,
              ones_v, zeros_v, c_sh):
    del tok_hbm
    core = lax.axis_index("c")
    sub = lax.axis_index("s")
    wid = sub * NC + core

    start = wid * nbc + 8 * jnp.minimum(wid, n8c)
    pltpu.sync_copy(c1_hbm.at[pl.ds(start, nbc)], c1_v.at[pl.ds(0, nbc)])
    pltpu.sync_copy(c2_hbm.at[pl.ds(start, nbc)], c2_v.at[pl.ds(0, nbc)])

    @pl.when(wid < n8c)
    def _():
        pltpu.sync_copy(c1_hbm.at[pl.ds(start + nbc, 8)],
                        c1_v.at[pl.ds(nbc, 8)])
        pltpu.sync_copy(c2_hbm.at[pl.ds(start + nbc, 8)],
                        c2_v.at[pl.ds(nbc, 8)])

    if ragc:
        @pl.when(wid == NW - 1)
        def _():
            pltpu.sync_copy(c1_hbm.at[pl.ds(start + nbc, ragc)],
                            c1_v.at[pl.ds(nbc, ragc)])
            pltpu.sync_copy(c2_hbm.at[pl.ds(start + nbc, ragc)],
                            c2_v.at[pl.ds(nbc, ragc)])

    nb_w = (nbc + 8 * (wid < n8c).astype(jnp.int32)
            + ragc * (wid == NW - 1).astype(jnp.int32))

    zeros = jnp.zeros((LANES,), jnp.float32)
    ones = jnp.ones((LANES,), jnp.float32)
    for j in range(B // LANES):
        ones_v[pl.ds(j * LANES, LANES)] = ones

    zlen = B * CCOLS

    def _z(i, carry):
        zeros_v[pl.ds(i * LANES, LANES)] = zeros
        return carry
    lax.fori_loop(0, zlen // LANES, _z, 0)

    cbase = sub * (CROWS // NS) * CCOLS
    for r in range((CROWS // NS) * CCOLS // zlen):
        pltpu.sync_copy(zeros_v, c_sh.at[pl.ds(cbase + r * zlen, zlen)])
    plsc.subcore_barrier()

    def _batch(b, carry):
        pltpu.sync_copy(ones_v, c_sh.at[c1_v.at[b]], add=True)
        pltpu.sync_copy(ones_v, c_sh.at[c2_v.at[b]], add=True)
        return carry
    lax.fori_loop(0, nb_w, _batch, 0)
    plsc.subcore_barrier()

    clen = (CROWS // NS) * CCOLS
    pltpu.sync_copy(c_sh.at[pl.ds(cbase, clen)],
                    cp_hbm.at[core, pl.ds(cbase, clen)])


@functools.partial(jax.jit, static_argnames=("n_nodes",))
def _sc_scatter(x, eidx3, c13, c23, *, n_nodes):
    nb_t = eidx3.shape[1]
    acc_rows = -(-n_nodes // (NS * 8)) * (NS * 8)
    rows_per_tile = acc_rows // NS
    mesh = plsc.VectorSubcoreMesh(core_axis_name="c", subcore_axis_name="s",
                                  num_cores=NC, num_subcores=NS)

    nb0 = int(nb_t * SPLIT0 / NS / 8 + 0.5) * 8
    start1 = nb0 * NS
    rest = nb_t - start1
    base1 = (rest // NS) // 8 * 8
    n8, rag = divmod(rest - base1 * NS, 8)
    assert 0 <= n8 < NS - 1 and rest >= 0
    assert nb0 % 2 == 0 and base1 % 2 == 0 and rag % 2 == 0
    nbmax = max(nb0, base1 + 8, base1 + rag)

    acc = pl.kernel(
        functools.partial(_acc_body, nb0, base1, start1, n8, rag,
                          rows_per_tile),
        out_type=jax.ShapeDtypeStruct((NC, acc_rows, D), jnp.float32),
        mesh=mesh,
        scratch_types=(
            pltpu.VMEM((nbmax, B), jnp.int32),
            pltpu.VMEM((nbmax, B), jnp.int32),
            pltpu.VMEM((B, D), jnp.float32),
            pltpu.VMEM_SHARED((acc_rows, D), jnp.float32),
            pltpu.SemaphoreType.DMA,
        ),
    )(x, eidx3)

    tok = lax.slice(acc, (0, 0, 0), (1, 8, 8))

    nbc = (nb_t // NW) // 8 * 8
    n8c, ragc = divmod(nb_t - nbc * NW, 8)
    assert 0 <= n8c < NW - 1
    nbcmax = nbc + max(8, ragc)
    cp = pl.kernel(
        functools.partial(_cnt_body, nbc, n8c, ragc),
        out_type=jax.ShapeDtypeStruct((NC, CROWS * CCOLS), jnp.float32),
        mesh=mesh,
        scratch_types=(
            pltpu.VMEM((nbcmax, B), jnp.int32),
            pltpu.VMEM((nbcmax, B), jnp.int32),
            pltpu.VMEM((B,), jnp.float32),
            pltpu.VMEM((B * CCOLS,), jnp.float32),
            pltpu.VMEM_SHARED((CROWS * CCOLS,), jnp.float32),
        ),
    )(tok, c13, c23)
    return acc, cp


def _mlp_body(accp, cp, e, w1, b1, w2, b2, out_ref):
    acc = accp[0] + accp[1]
    cb = cp[0] + cp[1]
    aggr = acc + jnp.dot(cb, e[...], preferred_element_type=jnp.float32)
    h = jnp.maximum(
        jnp.dot(aggr, w1[...], preferred_element_type=jnp.float32) + b1[...],
        0.0)
    out_ref[...] = (
        jnp.dot(h, w2[...], preferred_element_type=jnp.float32) + b2[...])


@functools.partial(jax.jit, static_argnames=("n",))
def _tc_mlp(accp, cp, e, w1, b1, w2, b2, *, n):
    blk = 1000 if n % 1000 == 0 else n
    grid = n // blk
    return pl.pallas_call(
        _mlp_body,
        grid=(grid,),
        in_specs=[
            pl.BlockSpec((NC, blk, D), lambda i: (0, i, 0)),
            pl.BlockSpec((NC, blk, CCOLS), lambda i: (0, i, 0)),
            pl.BlockSpec(e.shape, lambda i: (0, 0)),
            pl.BlockSpec(w1.shape, lambda i: (0, 0)),
            pl.BlockSpec(b1.shape, lambda i: (0,)),
            pl.BlockSpec(w2.shape, lambda i: (0, 0)),
            pl.BlockSpec(b2.shape, lambda i: (0,)),
        ],
        out_specs=pl.BlockSpec((blk, D), lambda i: (i, 0)),
        out_shape=jax.ShapeDtypeStruct((n, D), jnp.float32),
    )(accp, cp, e, w1, b1, w2, b2)


def kernel(x, edge_index, edge_attr, edge_embedding1, edge_embedding2,
           W1, b1, W2, b2):
    n_nodes, d = x.shape
    n_edges = edge_index.shape[1]
    assert d == D
    assert n_edges % B == 0 and n_nodes <= CROWS

    eidx3 = edge_index.reshape(2, n_edges // B, B)
    c1 = (edge_index[1] * CCOLS + edge_attr[:, 0]).reshape(n_edges // B, B)
    c2 = (edge_index[1] * CCOLS + 6 + edge_attr[:, 1]).reshape(n_edges // B, B)

    accp, cflat = _sc_scatter(x, eidx3, c1, c2, n_nodes=n_nodes)
    cp = cflat.reshape(NC, CROWS, CCOLS)

    epad = jnp.concatenate(
        [edge_embedding1, edge_embedding2,
         jnp.zeros((CCOLS - edge_embedding1.shape[0] - edge_embedding2.shape[0],
                    D), jnp.float32)], axis=0)
    return _tc_mlp(accp, cp, epad, W1, b1, W2, b2, n=n_nodes)


# speedup vs baseline: 1.0626x; 1.0626x over previous
import functools

import jax
import jax.numpy as jnp
from jax import lax
from jax.experimental import pallas as pl
from jax.experimental.pallas import tpu as pltpu
from jax.experimental.pallas import tpu_sc as plsc

D = 128
LANES = 16
NC = 2
NS = 16
NW = NC * NS
B = 128
CCOLS = 16
CROWS = 10240
SPLIT0 = 0.51


def _acc_body(nb0, base1, start1, n8, rag, rows_per_tile, x_hbm, eidx_hbm,
              accp_hbm, src_v, dst_v, rowbuf, acc_sh, gsem):
    core = lax.axis_index("c")
    sub = lax.axis_index("s")

    s0 = sub * nb0
    s1 = start1 + sub * base1 + 8 * jnp.minimum(sub, n8)

    @pl.when(core == 0)
    def _():
        pltpu.sync_copy(eidx_hbm.at[0, pl.ds(s0, nb0)], src_v.at[pl.ds(0, nb0)])
        pltpu.sync_copy(eidx_hbm.at[1, pl.ds(s0, nb0)], dst_v.at[pl.ds(0, nb0)])

    @pl.when(core == 1)
    def _():
        pltpu.sync_copy(eidx_hbm.at[0, pl.ds(s1, base1)],
                        src_v.at[pl.ds(0, base1)])
        pltpu.sync_copy(eidx_hbm.at[1, pl.ds(s1, base1)],
                        dst_v.at[pl.ds(0, base1)])

    @pl.when((core == 1) & (sub < n8))
    def _():
        pltpu.sync_copy(eidx_hbm.at[0, pl.ds(s1 + base1, 8)],
                        src_v.at[pl.ds(base1, 8)])
        pltpu.sync_copy(eidx_hbm.at[1, pl.ds(s1 + base1, 8)],
                        dst_v.at[pl.ds(base1, 8)])

    if rag:
        @pl.when((core == 1) & (sub == NS - 1))
        def _():
            pltpu.sync_copy(eidx_hbm.at[0, pl.ds(s1 + base1, rag)],
                            src_v.at[pl.ds(base1, rag)])
            pltpu.sync_copy(eidx_hbm.at[1, pl.ds(s1 + base1, rag)],
                            dst_v.at[pl.ds(base1, rag)])

    nb_w = jnp.where(
        core == 0, nb0,
        base1 + 8 * (sub < n8).astype(jnp.int32)
        + rag * (sub == NS - 1).astype(jnp.int32))

    zeros = jnp.zeros((LANES,), jnp.float32)

    def _zrow(i, carry):
        for j in range(D // LANES):
            rowbuf[i, pl.ds(j * LANES, LANES)] = zeros
        return carry
    lax.fori_loop(0, B, _zrow, 0)

    base = sub * rows_per_tile
    nfull = rows_per_tile // B
    rem = rows_per_tile - nfull * B
    for r in range(nfull):
        pltpu.sync_copy(rowbuf, acc_sh.at[pl.ds(base + r * B, B)])
    if rem:
        pltpu.sync_copy(rowbuf.at[pl.ds(0, rem)],
                        acc_sh.at[pl.ds(base + nfull * B, rem)])
    plsc.subcore_barrier()

    def _batch(b, carry):
        pltpu.async_copy(x_hbm.at[src_v.at[b]], rowbuf, gsem).wait()
        pltpu.sync_copy(rowbuf, acc_sh.at[dst_v.at[b]], add=True)
        return carry
    lax.fori_loop(0, nb_w, _batch, 0)
    plsc.subcore_barrier()

    pltpu.sync_copy(acc_sh.at[pl.ds(base, rows_per_tile)],
                    accp_hbm.at[core, pl.ds(base, rows_per_tile)])


def _cnt_body(nbc, n8c, ragc, tok_hbm, c1_hbm, c2_hbm, cp_hbm, c1_v, c2_v,
              ones_v, zeros_v, c_sh):
    del tok_hbm
    core = lax.axis_index("c")
    sub = lax.axis_index("s")
    wid = sub * NC + core

    start = wid * nbc + 8 * jnp.minimum(wid, n8c)
    pltpu.sync_copy(c1_hbm.at[pl.ds(start, nbc)], c1_v.at[pl.ds(0, nbc)])
    pltpu.sync_copy(c2_hbm.at[pl.ds(start, nbc)], c2_v.at[pl.ds(0, nbc)])

    @pl.when(wid < n8c)
    def _():
        pltpu.sync_copy(c1_hbm.at[pl.ds(start + nbc, 8)],
                        c1_v.at[pl.ds(nbc, 8)])
        pltpu.sync_copy(c2_hbm.at[pl.ds(start + nbc, 8)],
                        c2_v.at[pl.ds(nbc, 8)])

    if ragc:
        @pl.when(wid == NW - 1)
        def _():
            pltpu.sync_copy(c1_hbm.at[pl.ds(start + nbc, ragc)],
                            c1_v.at[pl.ds(nbc, ragc)])
            pltpu.sync_copy(c2_hbm.at[pl.ds(start + nbc, ragc)],
                            c2_v.at[pl.ds(nbc, ragc)])

    nb_w = (nbc + 8 * (wid < n8c).astype(jnp.int32)
            + ragc * (wid == NW - 1).astype(jnp.int32))

    zeros = jnp.zeros((LANES,), jnp.float32)
    ones = jnp.ones((LANES,), jnp.float32)
    for j in range(B // LANES):
        ones_v[pl.ds(j * LANES, LANES)] = ones

    zlen = B * CCOLS

    def _z(i, carry):
        zeros_v[pl.ds(i * LANES, LANES)] = zeros
        return carry
    lax.fori_loop(0, zlen // LANES, _z, 0)

    cbase = sub * (CROWS // NS) * CCOLS
    for r in range((CROWS // NS) * CCOLS // zlen):
        pltpu.sync_copy(zeros_v, c_sh.at[pl.ds(cbase + r * zlen, zlen)])
    plsc.subcore_barrier()

    def _batch(b, carry):
        pltpu.sync_copy(ones_v, c_sh.at[c1_v.at[b]], add=True)
        pltpu.sync_copy(ones_v, c_sh.at[c2_v.at[b]], add=True)
        return carry
    lax.fori_loop(0, nb_w, _batch, 0)
    plsc.subcore_barrier()

    clen = (CROWS // NS) * CCOLS
    pltpu.sync_copy(c_sh.at[pl.ds(cbase, clen)],
                    cp_hbm.at[core, pl.ds(cbase, clen)])


@functools.partial(jax.jit, static_argnames=("n_nodes",))
def _sc_scatter(x, eidx3, c13, c23, *, n_nodes):
    nb_t = eidx3.shape[1]
    acc_rows = -(-n_nodes // (NS * 8)) * (NS * 8)
    rows_per_tile = acc_rows // NS
    mesh = plsc.VectorSubcoreMesh(core_axis_name="c", subcore_axis_name="s",
                                  num_cores=NC, num_subcores=NS)

    nb0 = int(nb_t * SPLIT0 / NS / 8 + 0.5) * 8
    start1 = nb0 * NS
    rest = nb_t - start1
    base1 = (rest // NS) // 8 * 8
    n8, rag = divmod(rest - base1 * NS, 8)
    assert 0 <= n8 < NS - 1 and rest >= 0
    assert nb0 % 2 == 0 and base1 % 2 == 0 and rag % 2 == 0
    nbmax = max(nb0, base1 + 8, base1 + rag)

    acc = pl.kernel(
        functools.partial(_acc_body, nb0, base1, start1, n8, rag,
                          rows_per_tile),
        out_type=jax.ShapeDtypeStruct((NC, acc_rows, D), jnp.float32),
        mesh=mesh,
        scratch_types=(
            pltpu.VMEM((nbmax, B), jnp.int32),
            pltpu.VMEM((nbmax, B), jnp.int32),
            pltpu.VMEM((B, D), jnp.float32),
            pltpu.VMEM_SHARED((acc_rows, D), jnp.float32),
            pltpu.SemaphoreType.DMA,
        ),
    )(x, eidx3)

    tok = lax.slice(acc, (0, 0, 0), (1, 8, 8))

    nbc = (nb_t // NW) // 8 * 8
    n8c, ragc = divmod(nb_t - nbc * NW, 8)
    assert 0 <= n8c < NW - 1
    nbcmax = nbc + max(8, ragc)
    cp = pl.kernel(
        functools.partial(_cnt_body, nbc, n8c, ragc),
        out_type=jax.ShapeDtypeStruct((NC, CROWS * CCOLS), jnp.float32),
        mesh=mesh,
        scratch_types=(
            pltpu.VMEM((nbcmax, B), jnp.int32),
            pltpu.VMEM((nbcmax, B), jnp.int32),
            pltpu.VMEM((B,), jnp.float32),
            pltpu.VMEM((B * CCOLS,), jnp.float32),
            pltpu.VMEM_SHARED((CROWS * CCOLS,), jnp.float32),
        ),
    )(tok, c13, c23)
    return acc, cp


def _mlp_body(accp, cp, e, w1, b1, w2, b2, out_ref):
    acc = accp[0] + accp[1]
    cb = cp[0] + cp[1]
    aggr = acc + jnp.dot(cb, e[...], preferred_element_type=jnp.float32)
    h = jnp.maximum(
        jnp.dot(aggr, w1[...], preferred_element_type=jnp.float32) + b1[...],
        0.0)
    out_ref[...] = (
        jnp.dot(h, w2[...], preferred_element_type=jnp.float32) + b2[...])


@functools.partial(jax.jit, static_argnames=("n",))
def _tc_mlp(accp, cp, e, w1, b1, w2, b2, *, n):
    blk = 1000 if n % 1000 == 0 else n
    grid = n // blk
    return pl.pallas_call(
        _mlp_body,
        grid=(grid,),
        in_specs=[
            pl.BlockSpec((NC, blk, D), lambda i: (0, i, 0)),
            pl.BlockSpec((NC, blk, CCOLS), lambda i: (0, i, 0)),
            pl.BlockSpec(e.shape, lambda i: (0, 0)),
            pl.BlockSpec(w1.shape, lambda i: (0, 0)),
            pl.BlockSpec(b1.shape, lambda i: (0,)),
            pl.BlockSpec(w2.shape, lambda i: (0, 0)),
            pl.BlockSpec(b2.shape, lambda i: (0,)),
        ],
        out_specs=pl.BlockSpec((blk, D), lambda i: (i, 0)),
        out_shape=jax.ShapeDtypeStruct((n, D), jnp.float32),
    )(accp, cp, e, w1, b1, w2, b2)


def kernel(x, edge_index, edge_attr, edge_embedding1, edge_embedding2,
           W1, b1, W2, b2):
    n_nodes, d = x.shape
    n_edges = edge_index.shape[1]
    assert d == D
    assert n_edges % B == 0 and n_nodes <= CROWS

    eidx3 = edge_index.reshape(2, n_edges // B, B)
    c1 = (edge_index[1] * CCOLS + edge_attr[:, 0]).reshape(n_edges // B, B)
    c2 = (edge_index[1] * CCOLS + 6 + edge_attr[:, 1]).reshape(n_edges // B, B)

    accp, cflat = _sc_scatter(x, eidx3, c1, c2, n_nodes=n_nodes)
    cp = cflat.reshape(NC, CROWS, CCOLS)

    epad = jnp.concatenate(
        [edge_embedding1, edge_embedding2,
         jnp.zeros((CCOLS - edge_embedding1.shape[0] - edge_embedding2.shape[0],
                    D), jnp.float32)], axis=0)
    return _tc_mlp(accp, cp, epad, W1, b1, W2, b2, n=n_nodes)


# speedup vs baseline: 1.2209x; 1.1490x over previous
import functools

import jax
import jax.numpy as jnp
from jax import lax
from jax.experimental import pallas as pl
from jax.experimental.pallas import tpu as pltpu
from jax.experimental.pallas import tpu_sc as plsc

D = 128
HD = D // 2
LANES = 16
NC = 2
NS = 16
NW = NC * NS
B = 128
CCOLS = 16
CROWS = 10240


def _acc_body(nbs, n8s, rags, rows_per_tile, xl_hbm, xr_hbm, eidx_hbm,
              accp_hbm, src_v, dst_v, rowbuf, rowbuf1, acc_sh,
              gsem, gsem1):
    core = lax.axis_index("c")
    sub = lax.axis_index("s")

    s0 = sub * nbs + 8 * jnp.minimum(sub, n8s)

    pltpu.sync_copy(eidx_hbm.at[0, pl.ds(s0, nbs)], src_v.at[pl.ds(0, nbs)])
    pltpu.sync_copy(eidx_hbm.at[1, pl.ds(s0, nbs)], dst_v.at[pl.ds(0, nbs)])

    @pl.when(sub < n8s)
    def _():
        pltpu.sync_copy(eidx_hbm.at[0, pl.ds(s0 + nbs, 8)],
                        src_v.at[pl.ds(nbs, 8)])
        pltpu.sync_copy(eidx_hbm.at[1, pl.ds(s0 + nbs, 8)],
                        dst_v.at[pl.ds(nbs, 8)])

    if rags:
        @pl.when(sub == NS - 1)
        def _():
            pltpu.sync_copy(eidx_hbm.at[0, pl.ds(s0 + nbs, rags)],
                            src_v.at[pl.ds(nbs, rags)])
            pltpu.sync_copy(eidx_hbm.at[1, pl.ds(s0 + nbs, rags)],
                            dst_v.at[pl.ds(nbs, rags)])

    nb_w = (nbs + 8 * (sub < n8s).astype(jnp.int32)
            + rags * (sub == NS - 1).astype(jnp.int32))

    zeros = jnp.zeros((LANES,), jnp.float32)

    def _zrow(i, carry):
        for j in range(HD // LANES):
            rowbuf[i, pl.ds(j * LANES, LANES)] = zeros
        return carry
    lax.fori_loop(0, B, _zrow, 0)

    base = sub * rows_per_tile
    nfull = rows_per_tile // B
    rem = rows_per_tile - nfull * B
    for r in range(nfull):
        pltpu.sync_copy(rowbuf, acc_sh.at[pl.ds(base + r * B, B)])
    if rem:
        pltpu.sync_copy(rowbuf.at[pl.ds(0, rem)],
                        acc_sh.at[pl.ds(base + nfull * B, rem)])
    plsc.subcore_barrier()

    def _gath(buf, b, sem):
        @pl.when(core == 0)
        def _():
            pltpu.async_copy(xl_hbm.at[src_v.at[b]], buf, sem)

        @pl.when(core == 1)
        def _():
            pltpu.async_copy(xr_hbm.at[src_v.at[b]], buf, sem)

    def _drain(buf, sem):
        pltpu.make_async_copy(xl_hbm.at[pl.ds(0, B)], buf, sem).wait()

    _gath(rowbuf, 0, gsem)

    def _batch2(k, carry):
        b0 = 2 * k
        _gath(rowbuf1, b0 + 1, gsem1)
        _drain(rowbuf, gsem)
        pltpu.sync_copy(rowbuf, acc_sh.at[dst_v.at[b0]], add=True)

        @pl.when(b0 + 2 < nb_w)
        def _():
            _gath(rowbuf, b0 + 2, gsem)
        _drain(rowbuf1, gsem1)
        pltpu.sync_copy(rowbuf1, acc_sh.at[dst_v.at[b0 + 1]], add=True)
        return carry
    lax.fori_loop(0, nb_w // 2, _batch2, 0)
    plsc.subcore_barrier()

    pltpu.sync_copy(acc_sh.at[pl.ds(base, rows_per_tile)],
                    accp_hbm.at[core, pl.ds(base, rows_per_tile)])


def _cnt_body(nbc, n8c, ragc, tok_hbm, c1_hbm, c2_hbm, cp_hbm, c1_v, c2_v,
              ones_v, zeros_v, c_sh):
    del tok_hbm
    core = lax.axis_index("c")
    sub = lax.axis_index("s")
    wid = sub * NC + core

    start = wid * nbc + 8 * jnp.minimum(wid, n8c)
    pltpu.sync_copy(c1_hbm.at[pl.ds(start, nbc)], c1_v.at[pl.ds(0, nbc)])
    pltpu.sync_copy(c2_hbm.at[pl.ds(start, nbc)], c2_v.at[pl.ds(0, nbc)])

    @pl.when(wid < n8c)
    def _():
        pltpu.sync_copy(c1_hbm.at[pl.ds(start + nbc, 8)],
                        c1_v.at[pl.ds(nbc, 8)])
        pltpu.sync_copy(c2_hbm.at[pl.ds(start + nbc, 8)],
                        c2_v.at[pl.ds(nbc, 8)])

    if ragc:
        @pl.when(wid == NW - 1)
        def _():
            pltpu.sync_copy(c1_hbm.at[pl.ds(start + nbc, ragc)],
                            c1_v.at[pl.ds(nbc, ragc)])
            pltpu.sync_copy(c2_hbm.at[pl.ds(start + nbc, ragc)],
                            c2_v.at[pl.ds(nbc, ragc)])

    nb_w = (nbc + 8 * (wid < n8c).astype(jnp.int32)
            + ragc * (wid == NW - 1).astype(jnp.int32))

    zeros = jnp.zeros((LANES,), jnp.float32)
    ones = jnp.ones((LANES,), jnp.float32)
    for j in range(B // LANES):
        ones_v[pl.ds(j * LANES, LANES)] = ones

    zlen = B * CCOLS

    def _z(i, carry):
        zeros_v[pl.ds(i * LANES, LANES)] = zeros
        return carry
    lax.fori_loop(0, zlen // LANES, _z, 0)

    cbase = sub * (CROWS // NS) * CCOLS
    for r in range((CROWS // NS) * CCOLS // zlen):
        pltpu.sync_copy(zeros_v, c_sh.at[pl.ds(cbase + r * zlen, zlen)])
    plsc.subcore_barrier()

    def _batch(b, carry):
        pltpu.sync_copy(ones_v, c_sh.at[c1_v.at[b]], add=True)
        pltpu.sync_copy(ones_v, c_sh.at[c2_v.at[b]], add=True)
        return carry
    lax.fori_loop(0, nb_w, _batch, 0)
    plsc.subcore_barrier()

    clen = (CROWS // NS) * CCOLS
    pltpu.sync_copy(c_sh.at[pl.ds(cbase, clen)],
                    cp_hbm.at[core, pl.ds(cbase, clen)])


@functools.partial(jax.jit, static_argnames=("n_nodes",))
def _sc_scatter(xl, xr, eidx3, c13, c23, *, n_nodes):
    nb_t = eidx3.shape[1]
    acc_rows = -(-n_nodes // (NS * 8)) * (NS * 8)
    rows_per_tile = acc_rows // NS
    mesh = plsc.VectorSubcoreMesh(core_axis_name="c", subcore_axis_name="s",
                                  num_cores=NC, num_subcores=NS)

    nbs = (nb_t // NS) // 8 * 8
    n8s, rags = divmod(nb_t - nbs * NS, 8)
    assert 0 <= n8s < NS - 1
    assert nbs % 2 == 0 and rags % 2 == 0
    nbmax = nbs + max(8, rags)

    acc = pl.kernel(
        functools.partial(_acc_body, nbs, n8s, rags, rows_per_tile),
        out_type=jax.ShapeDtypeStruct((NC, acc_rows, HD), jnp.float32),
        mesh=mesh,
        scratch_types=(
            pltpu.VMEM((nbmax, B), jnp.int32),
            pltpu.VMEM((nbmax, B), jnp.int32),
            pltpu.VMEM((B, HD), jnp.float32),
            pltpu.VMEM((B, HD), jnp.float32),
            pltpu.VMEM_SHARED((acc_rows, HD), jnp.float32),
            pltpu.SemaphoreType.DMA,
            pltpu.SemaphoreType.DMA,
        ),
        compiler_params=pltpu.CompilerParams(use_tc_tiling_on_sc=False),
    )(xl, xr, eidx3)

    tok = lax.slice(acc, (0, 0, 0), (1, 8, 8))

    nbc = (nb_t // NW) // 8 * 8
    n8c, ragc = divmod(nb_t - nbc * NW, 8)
    assert 0 <= n8c < NW - 1
    nbcmax = nbc + max(8, ragc)
    cp = pl.kernel(
        functools.partial(_cnt_body, nbc, n8c, ragc),
        out_type=jax.ShapeDtypeStruct((NC, CROWS * CCOLS), jnp.float32),
        mesh=mesh,
        scratch_types=(
            pltpu.VMEM((nbcmax, B), jnp.int32),
            pltpu.VMEM((nbcmax, B), jnp.int32),
            pltpu.VMEM((B,), jnp.float32),
            pltpu.VMEM((B * CCOLS,), jnp.float32),
            pltpu.VMEM_SHARED((CROWS * CCOLS,), jnp.float32),
        ),
    )(tok, c13, c23)
    return acc, cp


def _mlp_body(accp, cp, e, w1, b1, w2, b2, out_ref):
    acc = jnp.concatenate([accp[0], accp[1]], axis=1)
    cb = cp[0] + cp[1]
    aggr = acc + jnp.dot(cb, e[...], preferred_element_type=jnp.float32)
    h = jnp.maximum(
        jnp.dot(aggr, w1[...], preferred_element_type=jnp.float32) + b1[...],
        0.0)
    out_ref[...] = (
        jnp.dot(h, w2[...], preferred_element_type=jnp.float32) + b2[...])


@functools.partial(jax.jit, static_argnames=("n",))
def _tc_mlp(accp, cp, e, w1, b1, w2, b2, *, n):
    blk = 1000 if n % 1000 == 0 else n
    grid = n // blk
    return pl.pallas_call(
        _mlp_body,
        grid=(grid,),
        in_specs=[
            pl.BlockSpec((NC, blk, HD), lambda i: (0, i, 0)),
            pl.BlockSpec((NC, blk, CCOLS), lambda i: (0, i, 0)),
            pl.BlockSpec(e.shape, lambda i: (0, 0)),
            pl.BlockSpec(w1.shape, lambda i: (0, 0)),
            pl.BlockSpec(b1.shape, lambda i: (0,)),
            pl.BlockSpec(w2.shape, lambda i: (0, 0)),
            pl.BlockSpec(b2.shape, lambda i: (0,)),
        ],
        out_specs=pl.BlockSpec((blk, D), lambda i: (i, 0)),
        out_shape=jax.ShapeDtypeStruct((n, D), jnp.float32),
    )(accp, cp, e, w1, b1, w2, b2)


def kernel(x, edge_index, edge_attr, edge_embedding1, edge_embedding2,
           W1, b1, W2, b2):
    n_nodes, d = x.shape
    n_edges = edge_index.shape[1]
    assert d == D
    assert n_edges % B == 0 and n_nodes <= CROWS

    eidx3 = edge_index.reshape(2, n_edges // B, B)
    c1 = (edge_index[1] * CCOLS + edge_attr[:, 0]).reshape(n_edges // B, B)
    c2 = (edge_index[1] * CCOLS + 6 + edge_attr[:, 1]).reshape(n_edges // B, B)

    xl = x[:, :HD]
    xr = x[:, HD:]
    accp, cflat = _sc_scatter(xl, xr, eidx3, c1, c2, n_nodes=n_nodes)
    cp = cflat.reshape(NC, CROWS, CCOLS)

    epad = jnp.concatenate(
        [edge_embedding1, edge_embedding2,
         jnp.zeros((CCOLS - edge_embedding1.shape[0] - edge_embedding2.shape[0],
                    D), jnp.float32)], axis=0)
    return _tc_mlp(accp, cp, epad, W1, b1, W2, b2, n=n_nodes)


# speedup vs baseline: 1.2271x; 1.0051x over previous
import functools

import jax
import jax.numpy as jnp
from jax import lax
from jax.experimental import pallas as pl
from jax.experimental.pallas import tpu as pltpu
from jax.experimental.pallas import tpu_sc as plsc

D = 128
HD = D // 2
LANES = 16
NC = 2
NS = 16
NW = NC * NS
B = 128
CCOLS = 16
CROWS = 10240


def _acc_body(nbs, n8s, rags, rows_per_tile, xl_hbm, xr_hbm, eidx_hbm,
              accp_hbm, src_v, dst_v, rowbuf, rowbuf1, acc_sh,
              gsem, gsem1):
    core = lax.axis_index("c")
    sub = lax.axis_index("s")

    s0 = sub * nbs + 8 * jnp.minimum(sub, n8s)

    pltpu.sync_copy(eidx_hbm.at[0, pl.ds(s0, nbs)], src_v.at[pl.ds(0, nbs)])
    pltpu.sync_copy(eidx_hbm.at[1, pl.ds(s0, nbs)], dst_v.at[pl.ds(0, nbs)])

    @pl.when(sub < n8s)
    def _():
        pltpu.sync_copy(eidx_hbm.at[0, pl.ds(s0 + nbs, 8)],
                        src_v.at[pl.ds(nbs, 8)])
        pltpu.sync_copy(eidx_hbm.at[1, pl.ds(s0 + nbs, 8)],
                        dst_v.at[pl.ds(nbs, 8)])

    if rags:
        @pl.when(sub == NS - 1)
        def _():
            pltpu.sync_copy(eidx_hbm.at[0, pl.ds(s0 + nbs, rags)],
                            src_v.at[pl.ds(nbs, rags)])
            pltpu.sync_copy(eidx_hbm.at[1, pl.ds(s0 + nbs, rags)],
                            dst_v.at[pl.ds(nbs, rags)])

    nb_w = (nbs + 8 * (sub < n8s).astype(jnp.int32)
            + rags * (sub == NS - 1).astype(jnp.int32))

    zeros = jnp.zeros((LANES,), jnp.float32)

    def _zrow(i, carry):
        for j in range(HD // LANES):
            rowbuf[i, pl.ds(j * LANES, LANES)] = zeros
        return carry
    lax.fori_loop(0, B, _zrow, 0)

    base = sub * rows_per_tile
    nfull = rows_per_tile // B
    rem = rows_per_tile - nfull * B
    for r in range(nfull):
        pltpu.sync_copy(rowbuf, acc_sh.at[pl.ds(base + r * B, B)])
    if rem:
        pltpu.sync_copy(rowbuf.at[pl.ds(0, rem)],
                        acc_sh.at[pl.ds(base + nfull * B, rem)])
    plsc.subcore_barrier()

    def _gath(buf, b, sem):
        @pl.when(core == 0)
        def _():
            pltpu.async_copy(xl_hbm.at[src_v.at[b]], buf, sem)

        @pl.when(core == 1)
        def _():
            pltpu.async_copy(xr_hbm.at[src_v.at[b]], buf, sem)

    def _drain(buf, sem):
        pltpu.make_async_copy(xl_hbm.at[pl.ds(0, B)], buf, sem).wait()

    _gath(rowbuf, 0, gsem)

    def _batch2(k, carry):
        b0 = 2 * k
        _gath(rowbuf1, b0 + 1, gsem1)
        _drain(rowbuf, gsem)
        pltpu.sync_copy(rowbuf, acc_sh.at[dst_v.at[b0]], add=True)

        @pl.when(b0 + 2 < nb_w)
        def _():
            _gath(rowbuf, b0 + 2, gsem)
        _drain(rowbuf1, gsem1)
        pltpu.sync_copy(rowbuf1, acc_sh.at[dst_v.at[b0 + 1]], add=True)
        return carry
    lax.fori_loop(0, nb_w // 2, _batch2, 0)
    plsc.subcore_barrier()

    pltpu.sync_copy(acc_sh.at[pl.ds(base, rows_per_tile)],
                    accp_hbm.at[core, pl.ds(base, rows_per_tile)])


def _cnt_body(nbc, n8c, ragc, tok_hbm, c1_hbm, c2_hbm, cp_hbm, c1_v, c2_v,
              ones_v, zeros_v, c_sh):
    del tok_hbm
    core = lax.axis_index("c")
    sub = lax.axis_index("s")
    wid = sub * NC + core

    start = wid * nbc + 8 * jnp.minimum(wid, n8c)
    pltpu.sync_copy(c1_hbm.at[pl.ds(start, nbc)], c1_v.at[pl.ds(0, nbc)])
    pltpu.sync_copy(c2_hbm.at[pl.ds(start, nbc)], c2_v.at[pl.ds(0, nbc)])

    @pl.when(wid < n8c)
    def _():
        pltpu.sync_copy(c1_hbm.at[pl.ds(start + nbc, 8)],
                        c1_v.at[pl.ds(nbc, 8)])
        pltpu.sync_copy(c2_hbm.at[pl.ds(start + nbc, 8)],
                        c2_v.at[pl.ds(nbc, 8)])

    if ragc:
        @pl.when(wid == NW - 1)
        def _():
            pltpu.sync_copy(c1_hbm.at[pl.ds(start + nbc, ragc)],
                            c1_v.at[pl.ds(nbc, ragc)])
            pltpu.sync_copy(c2_hbm.at[pl.ds(start + nbc, ragc)],
                            c2_v.at[pl.ds(nbc, ragc)])

    nb_w = (nbc + 8 * (wid < n8c).astype(jnp.int32)
            + ragc * (wid == NW - 1).astype(jnp.int32))

    zeros = jnp.zeros((LANES,), jnp.float32)
    ones = jnp.ones((LANES,), jnp.float32)
    for j in range(B // LANES):
        ones_v[pl.ds(j * LANES, LANES)] = ones

    zlen = B * CCOLS

    def _z(i, carry):
        zeros_v[pl.ds(i * LANES, LANES)] = zeros
        return carry
    lax.fori_loop(0, zlen // LANES, _z, 0)

    cbase = sub * (CROWS // NS) * CCOLS
    for r in range((CROWS // NS) * CCOLS // zlen):
        pltpu.sync_copy(zeros_v, c_sh.at[pl.ds(cbase + r * zlen, zlen)])
    plsc.subcore_barrier()

    def _batch(b, carry):
        pltpu.sync_copy(ones_v, c_sh.at[c1_v.at[b]], add=True)
        pltpu.sync_copy(ones_v, c_sh.at[c2_v.at[b]], add=True)
        return carry
    lax.fori_loop(0, nb_w, _batch, 0)
    plsc.subcore_barrier()

    clen = (CROWS // NS) * CCOLS
    pltpu.sync_copy(c_sh.at[pl.ds(cbase, clen)],
                    cp_hbm.at[core, pl.ds(cbase, clen)])


@functools.partial(jax.jit, static_argnames=("n_nodes",))
def _sc_scatter(xl, xr, eidx3, c13, c23, *, n_nodes):
    nb_t = eidx3.shape[1]
    acc_rows = -(-n_nodes // (NS * 8)) * (NS * 8)
    rows_per_tile = acc_rows // NS
    mesh = plsc.VectorSubcoreMesh(core_axis_name="c", subcore_axis_name="s",
                                  num_cores=NC, num_subcores=NS)

    nbs = (nb_t // NS) // 8 * 8
    n8s, rags = divmod(nb_t - nbs * NS, 8)
    assert 0 <= n8s < NS - 1
    assert nbs % 2 == 0 and rags % 2 == 0
    nbmax = nbs + max(8, rags)

    acc = pl.kernel(
        functools.partial(_acc_body, nbs, n8s, rags, rows_per_tile),
        out_type=jax.ShapeDtypeStruct((NC, acc_rows, HD), jnp.float32),
        mesh=mesh,
        scratch_types=(
            pltpu.VMEM((nbmax, B), jnp.int32),
            pltpu.VMEM((nbmax, B), jnp.int32),
            pltpu.VMEM((B, HD), jnp.float32),
            pltpu.VMEM((B, HD), jnp.float32),
            pltpu.VMEM_SHARED((acc_rows, HD), jnp.float32),
            pltpu.SemaphoreType.DMA,
            pltpu.SemaphoreType.DMA,
        ),
        compiler_params=pltpu.CompilerParams(use_tc_tiling_on_sc=False),
    )(xl, xr, eidx3)

    tok = acc

    nbc = (nb_t // NW) // 8 * 8
    n8c, ragc = divmod(nb_t - nbc * NW, 8)
    assert 0 <= n8c < NW - 1
    nbcmax = nbc + max(8, ragc)
    cp = pl.kernel(
        functools.partial(_cnt_body, nbc, n8c, ragc),
        out_type=jax.ShapeDtypeStruct((NC, CROWS * CCOLS), jnp.float32),
        mesh=mesh,
        scratch_types=(
            pltpu.VMEM((nbcmax, B), jnp.int32),
            pltpu.VMEM((nbcmax, B), jnp.int32),
            pltpu.VMEM((B,), jnp.float32),
            pltpu.VMEM((B * CCOLS,), jnp.float32),
            pltpu.VMEM_SHARED((CROWS * CCOLS,), jnp.float32),
        ),
    )(tok, c13, c23)
    return acc, cp


def _mlp_body(accp, cp, e, w1, b1, w2, b2, out_ref):
    acc = jnp.concatenate([accp[0], accp[1]], axis=1)
    cb = cp[0] + cp[1]
    aggr = acc + jnp.dot(cb, e[...], preferred_element_type=jnp.float32)
    h = jnp.maximum(
        jnp.dot(aggr, w1[...], preferred_element_type=jnp.float32) + b1[...],
        0.0)
    out_ref[...] = (
        jnp.dot(h, w2[...], preferred_element_type=jnp.float32) + b2[...])


@functools.partial(jax.jit, static_argnames=("n",))
def _tc_mlp(accp, cp, e, w1, b1, w2, b2, *, n):
    blk = 1000 if n % 1000 == 0 else n
    grid = n // blk
    return pl.pallas_call(
        _mlp_body,
        grid=(grid,),
        in_specs=[
            pl.BlockSpec((NC, blk, HD), lambda i: (0, i, 0)),
            pl.BlockSpec((NC, blk, CCOLS), lambda i: (0, i, 0)),
            pl.BlockSpec(e.shape, lambda i: (0, 0)),
            pl.BlockSpec(w1.shape, lambda i: (0, 0)),
            pl.BlockSpec(b1.shape, lambda i: (0,)),
            pl.BlockSpec(w2.shape, lambda i: (0, 0)),
            pl.BlockSpec(b2.shape, lambda i: (0,)),
        ],
        out_specs=pl.BlockSpec((blk, D), lambda i: (i, 0)),
        out_shape=jax.ShapeDtypeStruct((n, D), jnp.float32),
    )(accp, cp, e, w1, b1, w2, b2)


def kernel(x, edge_index, edge_attr, edge_embedding1, edge_embedding2,
           W1, b1, W2, b2):
    n_nodes, d = x.shape
    n_edges = edge_index.shape[1]
    assert d == D
    assert n_edges % B == 0 and n_nodes <= CROWS

    eidx3 = edge_index.reshape(2, n_edges // B, B)
    c1 = (edge_index[1] * CCOLS + edge_attr[:, 0]).reshape(n_edges // B, B)
    c2 = (edge_index[1] * CCOLS + 6 + edge_attr[:, 1]).reshape(n_edges // B, B)

    xl = x[:, :HD]
    xr = x[:, HD:]
    accp, cflat = _sc_scatter(xl, xr, eidx3, c1, c2, n_nodes=n_nodes)
    cp = cflat.reshape(NC, CROWS, CCOLS)

    epad = jnp.concatenate(
        [edge_embedding1, edge_embedding2,
         jnp.zeros((CCOLS - edge_embedding1.shape[0] - edge_embedding2.shape[0],
                    D), jnp.float32)], axis=0)
    return _tc_mlp(accp, cp, epad, W1, b1, W2, b2, n=n_nodes)
